# baseline (speedup 1.0000x reference)
import functools

import jax
import jax.numpy as jnp
from jax import lax
from jax.experimental import pallas as pl
from jax.experimental.pallas import tpu as pltpu
from jax.experimental.pallas import tpu_sc as plsc

EMBED_DIM = 64


def kernel(indices, table):
    B0, S = indices.shape
    B = B0 * S
    D = table.shape[1]
    idx_flat = indices.reshape(B).astype(jnp.int32)

    info = plsc.get_sparse_core_info()
    NW = info.num_cores * info.num_subcores
    b_per_w = B // NW
    C = 800
    n_chunks = b_per_w // C

    mesh = plsc.VectorSubcoreMesh(core_axis_name="c", subcore_axis_name="s")

    @functools.partial(
        pl.kernel,
        mesh=mesh,
        compiler_params=pltpu.CompilerParams(use_tc_tiling_on_sc=False),
        out_type=jax.ShapeDtypeStruct((B, D), jnp.float32),
        scratch_types=[
            pltpu.VMEM((C,), jnp.int32),
            pltpu.VMEM((C, D), jnp.float32),
            pltpu.SemaphoreType.DMA,
        ],
    )
    def emb(idx_hbm, table_hbm, out_hbm, idx_v, rows_v, sem):
        wid = lax.axis_index("s") * info.num_cores + lax.axis_index("c")
        base = wid * b_per_w

        def body(i, carry):
            off = base + i * C
            pltpu.sync_copy(idx_hbm.at[pl.ds(off, C)], idx_v)
            pltpu.async_copy(table_hbm.at[idx_v], rows_v, sem).wait()
            pltpu.sync_copy(rows_v, out_hbm.at[pl.ds(off, C)])
            return carry

        lax.fori_loop(0, n_chunks, body, 0)

    out = emb(idx_flat, table)
    return out.reshape(B0, S, D)


# speedup vs baseline: 1.0174x; 1.0174x over previous
import functools

import jax
import jax.numpy as jnp
from jax import lax
from jax.experimental import pallas as pl
from jax.experimental.pallas import tpu as pltpu
from jax.experimental.pallas import tpu_sc as plsc


def kernel(indices, table):
    B0, S = indices.shape
    B = B0 * S
    D = table.shape[1]
    idx_flat = indices.reshape(B).astype(jnp.int32)

    info = plsc.get_sparse_core_info()
    NW = info.num_cores * info.num_subcores
    b_per_w = B // NW
    C = 800
    n_pairs = b_per_w // (2 * C)

    mesh = plsc.VectorSubcoreMesh(core_axis_name="c", subcore_axis_name="s")

    @functools.partial(
        pl.kernel,
        mesh=mesh,
        compiler_params=pltpu.CompilerParams(use_tc_tiling_on_sc=False),
        out_type=jax.ShapeDtypeStruct((B, D), jnp.float32),
        scratch_types=[
            pltpu.VMEM((b_per_w,), jnp.int32),
            pltpu.VMEM((C, D), jnp.float32),
            pltpu.VMEM((C, D), jnp.float32),
            pltpu.SemaphoreType.DMA,
            pltpu.SemaphoreType.DMA,
            pltpu.SemaphoreType.DMA,
            pltpu.SemaphoreType.DMA,
        ],
    )
    def emb(idx_hbm, table_hbm, out_hbm, idx_v, rows0, rows1, g0, g1, o0, o1):
        wid = lax.axis_index("s") * info.num_cores + lax.axis_index("c")
        base = wid * b_per_w
        pltpu.sync_copy(idx_hbm.at[pl.ds(base, b_per_w)], idx_v)

        def body(j, carry):
            a_off = 2 * j * C
            b_off = a_off + C

            @pl.when(j > 0)
            def _():
                pltpu.make_async_copy(
                    rows0, out_hbm.at[pl.ds(base, C)], o0
                ).wait()

            ga = pltpu.async_copy(
                table_hbm.at[idx_v.at[pl.ds(a_off, C)]], rows0, g0
            )

            @pl.when(j > 0)
            def _():
                pltpu.make_async_copy(
                    rows1, out_hbm.at[pl.ds(base, C)], o1
                ).wait()

            gb = pltpu.async_copy(
                table_hbm.at[idx_v.at[pl.ds(b_off, C)]], rows1, g1
            )

            ga.wait()
            pltpu.async_copy(rows0, out_hbm.at[pl.ds(base + a_off, C)], o0)
            gb.wait()
            pltpu.async_copy(rows1, out_hbm.at[pl.ds(base + b_off, C)], o1)
            return carry

        lax.fori_loop(0, n_pairs, body, 0)
        last = base + (2 * n_pairs - 2) * C
        pltpu.make_async_copy(rows0, out_hbm.at[pl.ds(last, C)], o0).wait()
        pltpu.make_async_copy(rows1, out_hbm.at[pl.ds(last + C, C)], o1).wait()

    out = emb(idx_flat, table)
    return out.reshape(B0, S, D)


# speedup vs baseline: 1.0234x; 1.0059x over previous
import functools

import jax
import jax.numpy as jnp
from jax import lax
from jax.experimental import pallas as pl
from jax.experimental.pallas import tpu as pltpu
from jax.experimental.pallas import tpu_sc as plsc

NBUF = 8


def kernel(indices, table):
    B0, S = indices.shape
    B = B0 * S
    D = table.shape[1]
    idx_flat = indices.reshape(B).astype(jnp.int32)

    info = plsc.get_sparse_core_info()
    NW = info.num_cores * info.num_subcores
    rows_per_w = B0 // NW
    b_per_w = rows_per_w * S
    n_iters = rows_per_w // NBUF

    mesh = plsc.VectorSubcoreMesh(core_axis_name="c", subcore_axis_name="s")

    @functools.partial(
        pl.kernel,
        mesh=mesh,
        compiler_params=pltpu.CompilerParams(use_tc_tiling_on_sc=False),
        out_type=jax.ShapeDtypeStruct((B0, S, D), jnp.float32),
        scratch_types=[
            pltpu.VMEM((b_per_w,), jnp.int32),
            [pltpu.VMEM((S, D), jnp.float32)] * NBUF,
            [pltpu.SemaphoreType.DMA] * NBUF,
            [pltpu.SemaphoreType.DMA] * NBUF,
        ],
    )
    def emb(idx_hbm, table_hbm, out_hbm, idx_v, rows, gsem, osem):
        wid = lax.axis_index("s") * info.num_cores + lax.axis_index("c")
        base_r = wid * rows_per_w
        base_i = wid * b_per_w
        pltpu.sync_copy(idx_hbm.at[pl.ds(base_i, b_per_w)], idx_v)

        def body(j, carry):
            gathers = []
            for b in range(NBUF):
                @pl.when(j > 0)
                def _(b=b):
                    pltpu.make_async_copy(
                        rows[b], out_hbm.at[base_r], osem[b]
                    ).wait()

                off = (j * NBUF + b) * S
                gathers.append(pltpu.async_copy(
                    table_hbm.at[idx_v.at[pl.ds(off, S)]], rows[b], gsem[b]
                ))
            for b in range(NBUF):
                gathers[b].wait()
                pltpu.async_copy(
                    rows[b], out_hbm.at[base_r + j * NBUF + b], osem[b]
                )
            return carry

        lax.fori_loop(0, n_iters, body, 0)
        for b in range(NBUF):
            pltpu.make_async_copy(rows[b], out_hbm.at[base_r], osem[b]).wait()

    return emb(idx_flat, table)


# speedup vs baseline: 1.2420x; 1.2135x over previous
import functools

import jax
import jax.numpy as jnp
from jax import lax
from jax.experimental import pallas as pl
from jax.experimental.pallas import tpu as pltpu
from jax.experimental.pallas import tpu_sc as plsc


def kernel(indices, table):
    B0, S = indices.shape
    B = B0 * S
    V, D = table.shape
    DP = 128
    idx_flat = indices.reshape(B).astype(jnp.int32)
    table_p = jnp.pad(table, ((0, 0), (0, DP - D)))

    info = plsc.get_sparse_core_info()
    NW = info.num_cores * info.num_subcores
    b_per_w = B // NW
    C = 400
    n_pairs = b_per_w // (2 * C)

    mesh = plsc.VectorSubcoreMesh(core_axis_name="c", subcore_axis_name="s")

    @functools.partial(
        pl.kernel,
        mesh=mesh,
        compiler_params=pltpu.CompilerParams(use_tc_tiling_on_sc=True),
        out_type=jax.ShapeDtypeStruct((B, DP), jnp.float32),
        scratch_types=[
            pltpu.VMEM((b_per_w,), jnp.int32),
            pltpu.VMEM((C, DP), jnp.float32),
            pltpu.VMEM((C, DP), jnp.float32),
            pltpu.SemaphoreType.DMA,
            pltpu.SemaphoreType.DMA,
            pltpu.SemaphoreType.DMA,
            pltpu.SemaphoreType.DMA,
        ],
    )
    def emb(idx_hbm, table_hbm, out_hbm, idx_v, rows0, rows1, g0, g1, o0, o1):
        wid = lax.axis_index("s") * info.num_cores + lax.axis_index("c")
        base = wid * b_per_w
        pltpu.sync_copy(idx_hbm.at[pl.ds(base, b_per_w)], idx_v)

        def body(j, carry):
            a_off = 2 * j * C
            b_off = a_off + C

            @pl.when(j > 0)
            def _():
                pltpu.make_async_copy(
                    rows0, out_hbm.at[pl.ds(base, C)], o0
                ).wait()

            ga = pltpu.async_copy(
                table_hbm.at[idx_v.at[pl.ds(a_off, C)]], rows0, g0
            )

            @pl.when(j > 0)
            def _():
                pltpu.make_async_copy(
                    rows1, out_hbm.at[pl.ds(base, C)], o1
                ).wait()

            gb = pltpu.async_copy(
                table_hbm.at[idx_v.at[pl.ds(b_off, C)]], rows1, g1
            )

            ga.wait()
            pltpu.async_copy(rows0, out_hbm.at[pl.ds(base + a_off, C)], o0)
            gb.wait()
            pltpu.async_copy(rows1, out_hbm.at[pl.ds(base + b_off, C)], o1)
            return carry

        lax.fori_loop(0, n_pairs, body, 0)
        last = base + (2 * n_pairs - 2) * C
        pltpu.make_async_copy(rows0, out_hbm.at[pl.ds(last, C)], o0).wait()
        pltpu.make_async_copy(rows1, out_hbm.at[pl.ds(last + C, C)], o1).wait()

    out = emb(idx_flat, table_p)
    return out.reshape(B0, S, DP)[:, :, :D]


# speedup vs baseline: 1.2480x; 1.0049x over previous
import functools

import jax
import jax.numpy as jnp
from jax import lax
from jax.experimental import pallas as pl
from jax.experimental.pallas import tpu as pltpu
from jax.experimental.pallas import tpu_sc as plsc


def kernel(indices, table):
    B0, S = indices.shape
    B = B0 * S
    V, D = table.shape
    DP = 128
    idx_flat = indices.reshape(B).astype(jnp.int32)
    table_p = jnp.pad(table, ((0, 0), (0, DP - D)))

    info = plsc.get_sparse_core_info()
    NW = info.num_cores * info.num_subcores
    b_per_w = B // NW
    C = 200
    NBUF = 4
    n_iters = b_per_w // (NBUF * C)

    mesh = plsc.VectorSubcoreMesh(core_axis_name="c", subcore_axis_name="s")

    @functools.partial(
        pl.kernel,
        mesh=mesh,
        compiler_params=pltpu.CompilerParams(use_tc_tiling_on_sc=True),
        out_type=jax.ShapeDtypeStruct((B, DP), jnp.float32),
        scratch_types=[
            pltpu.VMEM((b_per_w,), jnp.int32),
            [pltpu.VMEM((C, DP), jnp.float32)] * NBUF,
            [pltpu.SemaphoreType.DMA] * NBUF,
            [pltpu.SemaphoreType.DMA] * NBUF,
        ],
    )
    def emb(idx_hbm, table_hbm, out_hbm, idx_v, rows, gsem, osem):
        wid = lax.axis_index("s") * info.num_cores + lax.axis_index("c")
        base = wid * b_per_w
        pltpu.sync_copy(idx_hbm.at[pl.ds(base, b_per_w)], idx_v)

        def body(j, carry):
            gathers = []
            for b in range(NBUF):
                @pl.when(j > 0)
                def _(b=b):
                    pltpu.make_async_copy(
                        rows[b], out_hbm.at[pl.ds(base, C)], osem[b]
                    ).wait()

                off = (j * NBUF + b) * C
                gathers.append(pltpu.async_copy(
                    table_hbm.at[idx_v.at[pl.ds(off, C)]], rows[b], gsem[b]
                ))
            for b in range(NBUF):
                off = (j * NBUF + b) * C
                gathers[b].wait()
                pltpu.async_copy(rows[b], out_hbm.at[pl.ds(base + off, C)], osem[b])
            return carry

        lax.fori_loop(0, n_iters, body, 0)
        for b in range(NBUF):
            pltpu.make_async_copy(
                rows[b], out_hbm.at[pl.ds(base, C)], osem[b]
            ).wait()

    out = emb(idx_flat, table_p)
    return out.reshape(B0, S, DP)[:, :, :D]
